# baseline (speedup 1.0000x reference)
import functools

import jax
import jax.numpy as jnp
from jax.experimental import pallas as pl
from jax.experimental.pallas import tpu as pltpu
from jax.experimental.pallas import tpu_sc as plsc

T = 4096
D = 1024
E = 16
K = 2
DFF = 512
NSH = 2

BT = 256
NB = (T * K) // BT + E
NPAD = NB * BT
NBM = NB + T // BT
BB = 1024

NC = 2
NS = 16
NW = NC * NS
A = K * T
APW = A // NW
C = 32
CH = APW // C


def _route_kernel(h_ref, gw_ref, w01_ref, rr_ref, blkmap_ref):
    x32 = h_ref[...]
    logits = jnp.dot(x32, gw_ref[...].T, preferred_element_type=jnp.float32)
    m = jnp.max(logits, axis=1, keepdims=True)
    ex = jnp.exp(logits - m)
    s = ex / jnp.sum(ex, axis=1, keepdims=True)
    iota = jax.lax.broadcasted_iota(jnp.int32, (T, E), 1)
    e1 = jnp.argmax(s, axis=1)
    m1 = iota == e1[:, None]
    e2 = jnp.argmax(jnp.where(m1, -jnp.inf, s), axis=1)
    m2 = iota == e2[:, None]
    w0 = jnp.sum(jnp.where(m1, s, 0.0), axis=1, keepdims=True)
    w1 = jnp.sum(jnp.where(m2, s, 0.0), axis=1, keepdims=True)
    w01_ref[...] = jnp.concatenate([w0, w1], axis=1)

    oh0 = m1.astype(jnp.int32)
    oh1 = m2.astype(jnp.int32)
    both = oh0 + oh1
    c = both
    for k in range(12):
        sh = 1 << k
        c = c + jnp.concatenate(
            [jnp.zeros((sh, E), jnp.int32), c[: T - sh]], axis=0)
    prior = c - both
    totals = jnp.sum(both, axis=0, keepdims=True)
    padded = ((totals + BT - 1) // BT) * BT
    ir = jax.lax.broadcasted_iota(jnp.int32, (E, E), 0)
    ic = jax.lax.broadcasted_iota(jnp.int32, (E, E), 1)
    tri = (ir < ic).astype(jnp.float32)
    base = jnp.dot(padded.astype(jnp.float32), tri,
                   preferred_element_type=jnp.float32).astype(jnp.int32)
    ends = base + padded

    rank0 = jnp.sum(prior * oh0, axis=1, keepdims=True)
    rank1 = jnp.sum(prior * oh1, axis=1, keepdims=True)
    base0 = jnp.sum(base * oh0, axis=1, keepdims=True)
    base1 = jnp.sum(base * oh1, axis=1, keepdims=True)
    rr_ref[...] = jnp.concatenate([rank0 + base0, rank1 + base1], axis=1)

    blk_start = jax.lax.broadcasted_iota(jnp.int32, (NBM, 1), 0) * BT
    blkmap_ref[...] = jnp.sum((blk_start >= ends).astype(jnp.int32), axis=1,
                              keepdims=True)


def _group_mlp_kernel(blkmap_ref, x_ref, h_ref, wg_ref, wu_ref, wd_ref,
                      swg_ref, swu_ref, swd_ref, os_ref, ysh_ref,
                      wg16, wu16, wd16, swg16, swu16, swd16):
    i = pl.program_id(0)
    eid = blkmap_ref[i]
    prev = blkmap_ref[jnp.maximum(i - 1, 0)]
    changed = jnp.logical_or(i == 0, eid != prev)

    @pl.when(jnp.logical_and(changed, eid < E))
    def _cast():
        wg16[...] = wg_ref[0].astype(jnp.bfloat16)
        wu16[...] = wu_ref[0].astype(jnp.bfloat16)
        wd16[...] = wd_ref[0].astype(jnp.bfloat16)

    @pl.when(eid < E)
    def _routed():
        x = x_ref[...].astype(jnp.bfloat16)
        g = jnp.dot(x, wg16[...], preferred_element_type=jnp.float32)
        u = jnp.dot(x, wu16[...], preferred_element_type=jnp.float32)
        a = (jax.nn.silu(g) * u).astype(jnp.bfloat16)
        os_ref[...] = jnp.dot(a, wd16[...], preferred_element_type=jnp.float32)

    @pl.when(i == 0)
    def _cast_shared():
        swg16[...] = swg_ref[...].astype(jnp.bfloat16)
        swu16[...] = swu_ref[...].astype(jnp.bfloat16)
        swd16[...] = swd_ref[...].astype(jnp.bfloat16)

    @pl.when(i >= NB)
    def _shared():
        x = h_ref[...].astype(jnp.bfloat16)
        g = jnp.dot(x, swg16[...], preferred_element_type=jnp.float32)
        u = jnp.dot(x, swu16[...], preferred_element_type=jnp.float32)
        a = (jax.nn.silu(g) * u).astype(jnp.bfloat16)
        ysh_ref[...] = jnp.dot(a, swd16[...], preferred_element_type=jnp.float32)


def _combine_kernel(ysh_ref, g0_ref, g1_ref, w01_ref, o_ref):
    w0 = w01_ref[:, 0:1]
    w1 = w01_ref[:, 1:2]
    o_ref[...] = ysh_ref[...] + w0 * g0_ref[...] + w1 * g1_ref[...]


def _vector_mesh():
    return plsc.VectorSubcoreMesh(core_axis_name="core",
                                  subcore_axis_name="subcore")


def _sc_dispatch(h, r3):
    @functools.partial(
        pl.kernel,
        out_type=jax.ShapeDtypeStruct((NPAD, D), jnp.float32),
        mesh=_vector_mesh(),
        scratch_types=[
            pltpu.VMEM((CH, C), jnp.int32),
            pltpu.VMEM((C, D), jnp.float32),
            pltpu.SemaphoreType.DMA,
        ])
    def disp(h_hbm, r_hbm, xs_hbm, idx_v, rows_v, sem):
        wid = jax.lax.axis_index("subcore") * NC + jax.lax.axis_index("core")
        pltpu.sync_copy(r_hbm.at[wid], idx_v)
        base = wid * APW

        @pl.loop(0, CH)
        def _(j):
            tok = jax.lax.rem(base + j * C, T)
            pltpu.sync_copy(h_hbm.at[pl.ds(tok, C)], rows_v)
            pltpu.async_copy(rows_v, xs_hbm.at[idx_v.at[j]], sem).wait()

    return disp(h, r3)


def _sc_collect(out_sorted, r3):
    @functools.partial(
        pl.kernel,
        out_type=jax.ShapeDtypeStruct((A, D), jnp.float32),
        mesh=_vector_mesh(),
        scratch_types=[
            pltpu.VMEM((CH, C), jnp.int32),
            pltpu.VMEM((C, D), jnp.float32),
            pltpu.SemaphoreType.DMA,
        ])
    def coll(os_hbm, r_hbm, g_hbm, idx_v, rows_v, sem):
        wid = jax.lax.axis_index("subcore") * NC + jax.lax.axis_index("core")
        pltpu.sync_copy(r_hbm.at[wid], idx_v)
        base = wid * APW

        @pl.loop(0, CH)
        def _(j):
            pltpu.async_copy(os_hbm.at[idx_v.at[j]], rows_v, sem).wait()
            pltpu.sync_copy(rows_v, g_hbm.at[pl.ds(base + j * C, C)])

    return coll(out_sorted, r3)


def kernel(h, gate_w, Wg, Wu, Wd, sWg, sWu, sWd):
    w01, rr, blkmap = pl.pallas_call(
        _route_kernel,
        in_specs=[
            pl.BlockSpec((T, D), lambda: (0, 0)),
            pl.BlockSpec((E, D), lambda: (0, 0)),
        ],
        out_specs=[
            pl.BlockSpec((T, K), lambda: (0, 0)),
            pl.BlockSpec((T, K), lambda: (0, 0)),
            pl.BlockSpec((NBM, 1), lambda: (0, 0)),
        ],
        out_shape=[
            jax.ShapeDtypeStruct((T, K), jnp.float32),
            jax.ShapeDtypeStruct((T, K), jnp.int32),
            jax.ShapeDtypeStruct((NBM, 1), jnp.int32),
        ],
    )(h, gate_w)

    r3 = rr.T.reshape(NW, CH, C)
    blkmap_flat = blkmap.reshape(NBM)

    x_sorted = _sc_dispatch(h, r3)

    grid_spec = pltpu.PrefetchScalarGridSpec(
        num_scalar_prefetch=1,
        grid=(NBM,),
        in_specs=[
            pl.BlockSpec((BT, D), lambda i, bm: (jnp.minimum(i, NB - 1), 0)),
            pl.BlockSpec((BT, D),
                         lambda i, bm: (jnp.maximum(i - NB, 0), 0)),
            pl.BlockSpec((1, D, DFF),
                         lambda i, bm: (jnp.minimum(bm[i], E - 1), 0, 0)),
            pl.BlockSpec((1, D, DFF),
                         lambda i, bm: (jnp.minimum(bm[i], E - 1), 0, 0)),
            pl.BlockSpec((1, DFF, D),
                         lambda i, bm: (jnp.minimum(bm[i], E - 1), 0, 0)),
            pl.BlockSpec((D, NSH * DFF), lambda i, bm: (0, 0)),
            pl.BlockSpec((D, NSH * DFF), lambda i, bm: (0, 0)),
            pl.BlockSpec((NSH * DFF, D), lambda i, bm: (0, 0)),
        ],
        out_specs=[
            pl.BlockSpec((BT, D), lambda i, bm: (jnp.minimum(i, NB - 1), 0)),
            pl.BlockSpec((BT, D),
                         lambda i, bm: (jnp.maximum(i - NB, 0), 0)),
        ],
        scratch_shapes=[
            pltpu.VMEM((D, DFF), jnp.bfloat16),
            pltpu.VMEM((D, DFF), jnp.bfloat16),
            pltpu.VMEM((DFF, D), jnp.bfloat16),
            pltpu.VMEM((D, NSH * DFF), jnp.bfloat16),
            pltpu.VMEM((D, NSH * DFF), jnp.bfloat16),
            pltpu.VMEM((NSH * DFF, D), jnp.bfloat16),
        ],
    )
    out_sorted, ysh = pl.pallas_call(
        _group_mlp_kernel,
        grid_spec=grid_spec,
        out_shape=[
            jax.ShapeDtypeStruct((NPAD, D), jnp.float32),
            jax.ShapeDtypeStruct((T, D), jnp.float32),
        ],
    )(blkmap_flat, x_sorted, h, Wg, Wu, Wd, sWg, sWu, sWd)

    g = _sc_collect(out_sorted, r3)

    nt = T // BB
    y = pl.pallas_call(
        _combine_kernel,
        grid=(nt,),
        in_specs=[
            pl.BlockSpec((BB, D), lambda t: (t, 0)),
            pl.BlockSpec((BB, D), lambda t: (t, 0)),
            pl.BlockSpec((BB, D), lambda t: (t + nt, 0)),
            pl.BlockSpec((BB, K), lambda t: (t, 0)),
        ],
        out_specs=pl.BlockSpec((BB, D), lambda t: (t, 0)),
        out_shape=jax.ShapeDtypeStruct((T, D), jnp.float32),
    )(ysh, g, g, w01)
    return y


# speedup vs baseline: 1.1813x; 1.1813x over previous
import functools

import jax
import jax.numpy as jnp
from jax.experimental import pallas as pl
from jax.experimental.pallas import tpu as pltpu
from jax.experimental.pallas import tpu_sc as plsc

T = 4096
D = 1024
E = 16
K = 2
DFF = 512
NSH = 2

BT = 512
NB = (T * K) // BT + E
NPAD = NB * BT
BB = 1024

NC = 2
NS = 16
NW = NC * NS
A = K * T
APW = A // NW
C = 32
CH = APW // C


def _route_kernel(h_ref, gw_ref, w01_ref, rr_ref, blkmap_ref):
    x32 = h_ref[...]
    logits = jnp.dot(x32, gw_ref[...].T, preferred_element_type=jnp.float32)
    m = jnp.max(logits, axis=1, keepdims=True)
    ex = jnp.exp(logits - m)
    s = ex / jnp.sum(ex, axis=1, keepdims=True)
    iota = jax.lax.broadcasted_iota(jnp.int32, (T, E), 1)
    e1 = jnp.argmax(s, axis=1)
    m1 = iota == e1[:, None]
    e2 = jnp.argmax(jnp.where(m1, -jnp.inf, s), axis=1)
    m2 = iota == e2[:, None]
    w0 = jnp.sum(jnp.where(m1, s, 0.0), axis=1, keepdims=True)
    w1 = jnp.sum(jnp.where(m2, s, 0.0), axis=1, keepdims=True)
    w01_ref[...] = jnp.concatenate([w0, w1], axis=1)

    oh0 = m1.astype(jnp.int32)
    oh1 = m2.astype(jnp.int32)
    both = oh0 + oh1
    c = both
    for k in range(12):
        sh = 1 << k
        c = c + jnp.concatenate(
            [jnp.zeros((sh, E), jnp.int32), c[: T - sh]], axis=0)
    prior = c - both
    totals = jnp.sum(both, axis=0, keepdims=True)
    padded = ((totals + BT - 1) // BT) * BT
    ir = jax.lax.broadcasted_iota(jnp.int32, (E, E), 0)
    ic = jax.lax.broadcasted_iota(jnp.int32, (E, E), 1)
    tri = (ir < ic).astype(jnp.float32)
    base = jnp.dot(padded.astype(jnp.float32), tri,
                   preferred_element_type=jnp.float32).astype(jnp.int32)
    ends = base + padded

    rank0 = jnp.sum(prior * oh0, axis=1, keepdims=True)
    rank1 = jnp.sum(prior * oh1, axis=1, keepdims=True)
    base0 = jnp.sum(base * oh0, axis=1, keepdims=True)
    base1 = jnp.sum(base * oh1, axis=1, keepdims=True)
    rr_ref[...] = jnp.concatenate([rank0 + base0, rank1 + base1], axis=1)

    blk_start = jax.lax.broadcasted_iota(jnp.int32, (NB, 1), 0) * BT
    blkmap_ref[...] = jnp.sum((blk_start >= ends).astype(jnp.int32), axis=1,
                              keepdims=True)


def _shared_kernel(x_ref, swg_ref, swu_ref, swd_ref, o_ref,
                   swg16, swu16, swd16):
    @pl.when(pl.program_id(0) == 0)
    def _cast():
        swg16[...] = swg_ref[...].astype(jnp.bfloat16)
        swu16[...] = swu_ref[...].astype(jnp.bfloat16)
        swd16[...] = swd_ref[...].astype(jnp.bfloat16)

    x = x_ref[...].astype(jnp.bfloat16)
    g = jnp.dot(x, swg16[...], preferred_element_type=jnp.float32)
    u = jnp.dot(x, swu16[...], preferred_element_type=jnp.float32)
    a = (jax.nn.silu(g) * u).astype(jnp.bfloat16)
    o_ref[...] = jnp.dot(a, swd16[...], preferred_element_type=jnp.float32)


def _group_mlp_kernel(blkmap_ref, x_ref, wg_ref, wu_ref, wd_ref, o_ref,
                      wg16, wu16, wd16):
    i = pl.program_id(0)
    eid = blkmap_ref[i]
    prev = blkmap_ref[jnp.maximum(i - 1, 0)]
    changed = jnp.logical_or(i == 0, eid != prev)

    @pl.when(jnp.logical_and(changed, eid < E))
    def _cast():
        wg16[...] = wg_ref[0].astype(jnp.bfloat16)
        wu16[...] = wu_ref[0].astype(jnp.bfloat16)
        wd16[...] = wd_ref[0].astype(jnp.bfloat16)

    @pl.when(eid < E)
    def _():
        x = x_ref[...].astype(jnp.bfloat16)
        g = jnp.dot(x, wg16[...], preferred_element_type=jnp.float32)
        u = jnp.dot(x, wu16[...], preferred_element_type=jnp.float32)
        a = (jax.nn.silu(g) * u).astype(jnp.bfloat16)
        o_ref[...] = jnp.dot(a, wd16[...], preferred_element_type=jnp.float32)


def _combine_kernel(ysh_ref, g0_ref, g1_ref, w01_ref, o_ref):
    w0 = w01_ref[:, 0:1]
    w1 = w01_ref[:, 1:2]
    o_ref[...] = ysh_ref[...] + w0 * g0_ref[...] + w1 * g1_ref[...]


def _vector_mesh():
    return plsc.VectorSubcoreMesh(core_axis_name="core",
                                  subcore_axis_name="subcore")


def _sc_dispatch(h, r3):
    @functools.partial(
        pl.kernel,
        out_type=jax.ShapeDtypeStruct((NPAD, D), jnp.float32),
        mesh=_vector_mesh(),
        scratch_types=[
            pltpu.VMEM((CH, C), jnp.int32),
            pltpu.VMEM((C, D), jnp.float32),
            pltpu.SemaphoreType.DMA,
        ])
    def disp(h_hbm, r_hbm, xs_hbm, idx_v, rows_v, sem):
        wid = jax.lax.axis_index("subcore") * NC + jax.lax.axis_index("core")
        pltpu.sync_copy(r_hbm.at[wid], idx_v)
        base = wid * APW

        @pl.loop(0, CH)
        def _(j):
            tok = jax.lax.rem(base + j * C, T)
            pltpu.sync_copy(h_hbm.at[pl.ds(tok, C)], rows_v)
            pltpu.async_copy(rows_v, xs_hbm.at[idx_v.at[j]], sem).wait()

    return disp(h, r3)


def _sc_collect(out_sorted, r3):
    @functools.partial(
        pl.kernel,
        out_type=jax.ShapeDtypeStruct((A, D), jnp.float32),
        mesh=_vector_mesh(),
        scratch_types=[
            pltpu.VMEM((CH, C), jnp.int32),
            pltpu.VMEM((C, D), jnp.float32),
            pltpu.SemaphoreType.DMA,
        ])
    def coll(os_hbm, r_hbm, g_hbm, idx_v, rows_v, sem):
        wid = jax.lax.axis_index("subcore") * NC + jax.lax.axis_index("core")
        pltpu.sync_copy(r_hbm.at[wid], idx_v)
        base = wid * APW

        @pl.loop(0, CH)
        def _(j):
            pltpu.async_copy(os_hbm.at[idx_v.at[j]], rows_v, sem).wait()
            pltpu.sync_copy(rows_v, g_hbm.at[pl.ds(base + j * C, C)])

    return coll(out_sorted, r3)


def kernel(h, gate_w, Wg, Wu, Wd, sWg, sWu, sWd):
    w01, rr, blkmap = pl.pallas_call(
        _route_kernel,
        in_specs=[
            pl.BlockSpec((T, D), lambda: (0, 0)),
            pl.BlockSpec((E, D), lambda: (0, 0)),
        ],
        out_specs=[
            pl.BlockSpec((T, K), lambda: (0, 0)),
            pl.BlockSpec((T, K), lambda: (0, 0)),
            pl.BlockSpec((NB, 1), lambda: (0, 0)),
        ],
        out_shape=[
            jax.ShapeDtypeStruct((T, K), jnp.float32),
            jax.ShapeDtypeStruct((T, K), jnp.int32),
            jax.ShapeDtypeStruct((NB, 1), jnp.int32),
        ],
    )(h, gate_w)

    r3 = rr.T.reshape(NW, CH, C)
    blkmap_flat = blkmap.reshape(NB)

    x_sorted = _sc_dispatch(h, r3)

    ysh = pl.pallas_call(
        _shared_kernel,
        grid=(T // BB,),
        in_specs=[
            pl.BlockSpec((BB, D), lambda t: (t, 0)),
            pl.BlockSpec((D, NSH * DFF), lambda t: (0, 0)),
            pl.BlockSpec((D, NSH * DFF), lambda t: (0, 0)),
            pl.BlockSpec((NSH * DFF, D), lambda t: (0, 0)),
        ],
        out_specs=pl.BlockSpec((BB, D), lambda t: (t, 0)),
        out_shape=jax.ShapeDtypeStruct((T, D), jnp.float32),
        scratch_shapes=[
            pltpu.VMEM((D, NSH * DFF), jnp.bfloat16),
            pltpu.VMEM((D, NSH * DFF), jnp.bfloat16),
            pltpu.VMEM((NSH * DFF, D), jnp.bfloat16),
        ],
    )(h, sWg, sWu, sWd)

    grid_spec = pltpu.PrefetchScalarGridSpec(
        num_scalar_prefetch=1,
        grid=(NB,),
        in_specs=[
            pl.BlockSpec((BT, D), lambda i, bm: (i, 0)),
            pl.BlockSpec((1, D, DFF),
                         lambda i, bm: (jnp.minimum(bm[i], E - 1), 0, 0)),
            pl.BlockSpec((1, D, DFF),
                         lambda i, bm: (jnp.minimum(bm[i], E - 1), 0, 0)),
            pl.BlockSpec((1, DFF, D),
                         lambda i, bm: (jnp.minimum(bm[i], E - 1), 0, 0)),
        ],
        out_specs=pl.BlockSpec((BT, D), lambda i, bm: (i, 0)),
        scratch_shapes=[
            pltpu.VMEM((D, DFF), jnp.bfloat16),
            pltpu.VMEM((D, DFF), jnp.bfloat16),
            pltpu.VMEM((DFF, D), jnp.bfloat16),
        ],
    )
    out_sorted = pl.pallas_call(
        _group_mlp_kernel,
        grid_spec=grid_spec,
        out_shape=jax.ShapeDtypeStruct((NPAD, D), jnp.float32),
    )(blkmap_flat, x_sorted, Wg, Wu, Wd)

    g = _sc_collect(out_sorted, r3)

    nt = T // BB
    y = pl.pallas_call(
        _combine_kernel,
        grid=(nt,),
        in_specs=[
            pl.BlockSpec((BB, D), lambda t: (t, 0)),
            pl.BlockSpec((BB, D), lambda t: (t, 0)),
            pl.BlockSpec((BB, D), lambda t: (t + nt, 0)),
            pl.BlockSpec((BB, K), lambda t: (t, 0)),
        ],
        out_specs=pl.BlockSpec((BB, D), lambda t: (t, 0)),
        out_shape=jax.ShapeDtypeStruct((T, D), jnp.float32),
    )(ysh, g, g, w01)
    return y


# speedup vs baseline: 1.1833x; 1.0017x over previous
import functools

import jax
import jax.numpy as jnp
from jax.experimental import pallas as pl
from jax.experimental.pallas import tpu as pltpu
from jax.experimental.pallas import tpu_sc as plsc

T = 4096
D = 1024
E = 16
K = 2
DFF = 512
NSH = 2

BT = 512
NB = (T * K) // BT + E
NPAD = NB * BT
BB = 512

NC = 2
NS = 16
NW = NC * NS
A = K * T
APW = A // NW
C = 32
CH = APW // C


def _route_kernel(h_ref, gw_ref, w01_ref, rr_ref, blkmap_ref):
    x32 = h_ref[...]
    logits = jnp.dot(x32, gw_ref[...].T, preferred_element_type=jnp.float32)
    m = jnp.max(logits, axis=1, keepdims=True)
    ex = jnp.exp(logits - m)
    s = ex / jnp.sum(ex, axis=1, keepdims=True)
    iota = jax.lax.broadcasted_iota(jnp.int32, (T, E), 1)
    e1 = jnp.argmax(s, axis=1)
    m1 = iota == e1[:, None]
    e2 = jnp.argmax(jnp.where(m1, -jnp.inf, s), axis=1)
    m2 = iota == e2[:, None]
    w0 = jnp.sum(jnp.where(m1, s, 0.0), axis=1, keepdims=True)
    w1 = jnp.sum(jnp.where(m2, s, 0.0), axis=1, keepdims=True)
    w01_ref[...] = jnp.concatenate([w0, w1], axis=1)

    oh0 = m1.astype(jnp.int32)
    oh1 = m2.astype(jnp.int32)
    both = oh0 + oh1
    c = both
    for k in range(12):
        sh = 1 << k
        c = c + jnp.concatenate(
            [jnp.zeros((sh, E), jnp.int32), c[: T - sh]], axis=0)
    prior = c - both
    totals = jnp.sum(both, axis=0, keepdims=True)
    padded = ((totals + BT - 1) // BT) * BT
    ir = jax.lax.broadcasted_iota(jnp.int32, (E, E), 0)
    ic = jax.lax.broadcasted_iota(jnp.int32, (E, E), 1)
    tri = (ir < ic).astype(jnp.float32)
    base = jnp.dot(padded.astype(jnp.float32), tri,
                   preferred_element_type=jnp.float32).astype(jnp.int32)
    ends = base + padded

    rank0 = jnp.sum(prior * oh0, axis=1, keepdims=True)
    rank1 = jnp.sum(prior * oh1, axis=1, keepdims=True)
    base0 = jnp.sum(base * oh0, axis=1, keepdims=True)
    base1 = jnp.sum(base * oh1, axis=1, keepdims=True)
    rr_ref[...] = jnp.concatenate([rank0 + base0, rank1 + base1], axis=1)

    blk_start = jax.lax.broadcasted_iota(jnp.int32, (NB, 1), 0) * BT
    blkmap_ref[...] = jnp.sum((blk_start >= ends).astype(jnp.int32), axis=1,
                              keepdims=True)


def _shared_kernel(x_ref, swg_ref, swu_ref, swd_ref, o_ref,
                   swg16, swu16, swd16):
    @pl.when(pl.program_id(0) == 0)
    def _cast():
        swg16[...] = swg_ref[...].astype(jnp.bfloat16)
        swu16[...] = swu_ref[...].astype(jnp.bfloat16)
        swd16[...] = swd_ref[...].astype(jnp.bfloat16)

    x = x_ref[...].astype(jnp.bfloat16)
    g = jnp.dot(x, swg16[...], preferred_element_type=jnp.float32)
    u = jnp.dot(x, swu16[...], preferred_element_type=jnp.float32)
    a = (jax.nn.silu(g) * u).astype(jnp.bfloat16)
    o_ref[...] = jnp.dot(a, swd16[...], preferred_element_type=jnp.float32)


def _group_mlp_kernel(blkmap_ref, x_ref, wg_ref, wu_ref, wd_ref, o_ref,
                      wg16, wu16, wd16):
    i = pl.program_id(0)
    eid = blkmap_ref[i]
    prev = blkmap_ref[jnp.maximum(i - 1, 0)]
    changed = jnp.logical_or(i == 0, eid != prev)

    @pl.when(jnp.logical_and(changed, eid < E))
    def _cast():
        wg16[...] = wg_ref[0].astype(jnp.bfloat16)
        wu16[...] = wu_ref[0].astype(jnp.bfloat16)
        wd16[...] = wd_ref[0].astype(jnp.bfloat16)

    @pl.when(eid < E)
    def _():
        x = x_ref[...].astype(jnp.bfloat16)
        g = jnp.dot(x, wg16[...], preferred_element_type=jnp.float32)
        u = jnp.dot(x, wu16[...], preferred_element_type=jnp.float32)
        a = (jax.nn.silu(g) * u).astype(jnp.bfloat16)
        o_ref[...] = jnp.dot(a, wd16[...], preferred_element_type=jnp.float32)


def _combine_kernel(ysh_ref, g0_ref, g1_ref, w01_ref, o_ref):
    w0 = w01_ref[:, 0:1]
    w1 = w01_ref[:, 1:2]
    o_ref[...] = ysh_ref[...] + w0 * g0_ref[...] + w1 * g1_ref[...]


def _vector_mesh():
    return plsc.VectorSubcoreMesh(core_axis_name="core",
                                  subcore_axis_name="subcore")


def _sc_dispatch(h, r3):
    @functools.partial(
        pl.kernel,
        out_type=jax.ShapeDtypeStruct((NPAD, D), jnp.float32),
        mesh=_vector_mesh(),
        scratch_types=[
            pltpu.VMEM((CH, C), jnp.int32),
            pltpu.VMEM((C, D), jnp.float32),
            pltpu.SemaphoreType.DMA,
        ])
    def disp(h_hbm, r_hbm, xs_hbm, idx_v, rows_v, sem):
        wid = jax.lax.axis_index("subcore") * NC + jax.lax.axis_index("core")
        pltpu.sync_copy(r_hbm.at[wid], idx_v)
        base = wid * APW

        @pl.loop(0, CH)
        def _(j):
            tok = jax.lax.rem(base + j * C, T)
            pltpu.sync_copy(h_hbm.at[pl.ds(tok, C)], rows_v)
            pltpu.async_copy(rows_v, xs_hbm.at[idx_v.at[j]], sem).wait()

    return disp(h, r3)


def _sc_collect(out_sorted, r3):
    @functools.partial(
        pl.kernel,
        out_type=jax.ShapeDtypeStruct((A, D), jnp.float32),
        mesh=_vector_mesh(),
        scratch_types=[
            pltpu.VMEM((CH, C), jnp.int32),
            pltpu.VMEM((C, D), jnp.float32),
            pltpu.SemaphoreType.DMA,
        ])
    def coll(os_hbm, r_hbm, g_hbm, idx_v, rows_v, sem):
        wid = jax.lax.axis_index("subcore") * NC + jax.lax.axis_index("core")
        pltpu.sync_copy(r_hbm.at[wid], idx_v)
        base = wid * APW

        @pl.loop(0, CH)
        def _(j):
            pltpu.async_copy(os_hbm.at[idx_v.at[j]], rows_v, sem).wait()
            pltpu.sync_copy(rows_v, g_hbm.at[pl.ds(base + j * C, C)])

    return coll(out_sorted, r3)


def kernel(h, gate_w, Wg, Wu, Wd, sWg, sWu, sWd):
    w01, rr, blkmap = pl.pallas_call(
        _route_kernel,
        in_specs=[
            pl.BlockSpec((T, D), lambda: (0, 0)),
            pl.BlockSpec((E, D), lambda: (0, 0)),
        ],
        out_specs=[
            pl.BlockSpec((T, K), lambda: (0, 0)),
            pl.BlockSpec((T, K), lambda: (0, 0)),
            pl.BlockSpec((NB, 1), lambda: (0, 0)),
        ],
        out_shape=[
            jax.ShapeDtypeStruct((T, K), jnp.float32),
            jax.ShapeDtypeStruct((T, K), jnp.int32),
            jax.ShapeDtypeStruct((NB, 1), jnp.int32),
        ],
    )(h, gate_w)

    r3 = rr.T.reshape(NW, CH, C)
    blkmap_flat = blkmap.reshape(NB)

    x_sorted = _sc_dispatch(h, r3)

    ysh = pl.pallas_call(
        _shared_kernel,
        grid=(T // BB,),
        in_specs=[
            pl.BlockSpec((BB, D), lambda t: (t, 0)),
            pl.BlockSpec((D, NSH * DFF), lambda t: (0, 0)),
            pl.BlockSpec((D, NSH * DFF), lambda t: (0, 0)),
            pl.BlockSpec((NSH * DFF, D), lambda t: (0, 0)),
        ],
        out_specs=pl.BlockSpec((BB, D), lambda t: (t, 0)),
        out_shape=jax.ShapeDtypeStruct((T, D), jnp.float32),
        scratch_shapes=[
            pltpu.VMEM((D, NSH * DFF), jnp.bfloat16),
            pltpu.VMEM((D, NSH * DFF), jnp.bfloat16),
            pltpu.VMEM((NSH * DFF, D), jnp.bfloat16),
        ],
    )(h, sWg, sWu, sWd)

    grid_spec = pltpu.PrefetchScalarGridSpec(
        num_scalar_prefetch=1,
        grid=(NB,),
        in_specs=[
            pl.BlockSpec((BT, D), lambda i, bm: (i, 0)),
            pl.BlockSpec((1, D, DFF),
                         lambda i, bm: (jnp.minimum(bm[i], E - 1), 0, 0)),
            pl.BlockSpec((1, D, DFF),
                         lambda i, bm: (jnp.minimum(bm[i], E - 1), 0, 0)),
            pl.BlockSpec((1, DFF, D),
                         lambda i, bm: (jnp.minimum(bm[i], E - 1), 0, 0)),
        ],
        out_specs=pl.BlockSpec((BT, D), lambda i, bm: (i, 0)),
        scratch_shapes=[
            pltpu.VMEM((D, DFF), jnp.bfloat16),
            pltpu.VMEM((D, DFF), jnp.bfloat16),
            pltpu.VMEM((DFF, D), jnp.bfloat16),
        ],
    )
    out_sorted = pl.pallas_call(
        _group_mlp_kernel,
        grid_spec=grid_spec,
        out_shape=jax.ShapeDtypeStruct((NPAD, D), jnp.float32),
    )(blkmap_flat, x_sorted, Wg, Wu, Wd)

    g = _sc_collect(out_sorted, r3)

    nt = T // BB
    y = pl.pallas_call(
        _combine_kernel,
        grid=(nt,),
        in_specs=[
            pl.BlockSpec((BB, D), lambda t: (t, 0)),
            pl.BlockSpec((BB, D), lambda t: (t, 0)),
            pl.BlockSpec((BB, D), lambda t: (t + nt, 0)),
            pl.BlockSpec((BB, K), lambda t: (t, 0)),
        ],
        out_specs=pl.BlockSpec((BB, D), lambda t: (t, 0)),
        out_shape=jax.ShapeDtypeStruct((T, D), jnp.float32),
    )(ysh, g, g, w01)
    return y
